# Initial kernel scaffold; baseline (speedup 1.0000x reference)
import jax
import jax.numpy as jnp
from jax.experimental import pallas as pl


def kernel(pos, edge_index):
    raise NotImplementedError("write your pallas kernel here")


# speedup vs baseline: 6.3887x; 6.3887x over previous
import functools
import math

import jax
import jax.numpy as jnp
from jax import lax
from jax.experimental import pallas as pl
from jax.experimental.pallas import tpu as pltpu
from jax.experimental.pallas import tpu_sc as plsc

N_NODES = 100000
N_EDGES = 3200000

NC, NS, L = 2, 16, 16
NW = NC * NS
EPT = N_EDGES // NW
CH13 = 2000
CH4 = 800

SQ3 = math.sqrt(3.0)
SQ5 = math.sqrt(5.0)
SQ7 = math.sqrt(7.0)
SQ15 = math.sqrt(15.0)
C3A = math.sqrt(42.0) / 6.0
C3B = math.sqrt(168.0) / 8.0
C3C = math.sqrt(7.0) / 2.0


def _rsqrt16(r2):
    i = plsc.bitcast(r2, jnp.int32)
    i = jnp.int32(0x5F3759DF) - lax.shift_right_logical(i, 1)
    y = plsc.bitcast(i, jnp.float32)
    for _ in range(3):
        y = y * (1.5 - 0.5 * r2 * y * y)
    return y


def _body(xcol, ycol, zcol, src, dst, vec_out, sh_out, comp, xtab,
          sbuf, dbuf, cbuf, vxb, vyb, vzb, vecb, shb):
    wid = lax.axis_index("s") * NC + lax.axis_index("c")
    base = wid * EPT
    iota = lax.iota(jnp.int32, L)

    for c, col in enumerate((xcol, ycol, zcol)):
        pltpu.sync_copy(col, xtab)

        def chunk13(k, _, c=c):
            e0 = pl.multiple_of(base + k * CH13, 8)
            pltpu.sync_copy(src.at[pl.ds(e0, CH13)], sbuf)
            pltpu.sync_copy(dst.at[pl.ds(e0, CH13)], dbuf)

            def grp(g, _):
                sl = pl.ds(g * L, L)
                si = sbuf[sl]
                di = dbuf[sl]
                xs = plsc.load_gather(xtab, [si])
                xd = plsc.load_gather(xtab, [di])
                cbuf[sl] = xd - xs
                return 0

            lax.fori_loop(0, CH13 // L, grp, 0)
            pltpu.sync_copy(cbuf, comp.at[pl.ds(c * N_EDGES + e0, CH13)])
            return 0

        lax.fori_loop(0, EPT // CH13, chunk13, 0)

    def chunk4(k, _):
        e0 = pl.multiple_of(base + k * CH4, 8)
        pltpu.sync_copy(comp.at[pl.ds(e0, CH4)], vxb)
        pltpu.sync_copy(comp.at[pl.ds(N_EDGES + e0, CH4)], vyb)
        pltpu.sync_copy(comp.at[pl.ds(2 * N_EDGES + e0, CH4)], vzb)

        def grp(g, _):
            sl = pl.ds(g * L, L)
            x = vxb[sl]
            y = vyb[sl]
            z = vzb[sl]
            rows = iota + g * L

            r2 = x * x + y * y + z * z
            rn = _rsqrt16(jnp.maximum(r2, 1e-24))
            ux = x * rn
            uy = y * rn
            uz = z * rn

            x2 = ux * ux
            y2 = uy * uy
            z2 = uz * uz
            x2z2 = x2 + z2
            sh20 = SQ15 * ux * uz
            sh21 = SQ15 * ux * uy
            sh22 = SQ5 * (y2 - 0.5 * x2z2)
            sh23 = SQ15 * uy * uz
            sh24 = (SQ15 / 2.0) * (z2 - x2)
            fy = 4.0 * y2 - x2z2
            sh = (
                jnp.full((L,), 1.0, jnp.float32),
                SQ3 * ux, SQ3 * uy, SQ3 * uz,
                sh20, sh21, sh22, sh23, sh24,
                C3A * (sh20 * uz + sh24 * ux),
                SQ7 * sh20 * uy,
                C3B * fy * ux,
                C3C * uy * (2.0 * y2 - 3.0 * x2z2),
                C3B * uz * fy,
                SQ7 * sh24 * uy,
                C3A * (sh24 * uz - sh20 * ux),
            )
            rows3 = rows * 3
            for ci, val in enumerate((x, y, z)):
                plsc.store_scatter(vecb, [rows3 + ci], val)
            rows16 = rows * 16
            for k16, val in enumerate(sh):
                plsc.store_scatter(shb, [rows16 + k16], val)
            return 0

        lax.fori_loop(0, CH4 // L, grp, 0)
        pltpu.sync_copy(vecb, vec_out.at[pl.ds(pl.multiple_of(e0 * 3, 8), CH4 * 3)])
        pltpu.sync_copy(shb, sh_out.at[pl.ds(pl.multiple_of(e0 * 16, 8), CH4 * 16)])
        return 0

    lax.fori_loop(0, EPT // CH4, chunk4, 0)


_sc_call = functools.partial(
    pl.kernel,
    mesh=plsc.VectorSubcoreMesh(core_axis_name="c", subcore_axis_name="s"),
    compiler_params=pltpu.CompilerParams(needs_layout_passes=False),
    out_type=[
        jax.ShapeDtypeStruct((N_EDGES * 3,), jnp.float32),
        jax.ShapeDtypeStruct((N_EDGES * 16,), jnp.float32),
        jax.ShapeDtypeStruct((N_EDGES * 3,), jnp.float32),
    ],
    scratch_types=[
        pltpu.VMEM((N_NODES,), jnp.float32),
        pltpu.VMEM((CH13,), jnp.int32),
        pltpu.VMEM((CH13,), jnp.int32),
        pltpu.VMEM((CH13,), jnp.float32),
        pltpu.VMEM((CH4,), jnp.float32),
        pltpu.VMEM((CH4,), jnp.float32),
        pltpu.VMEM((CH4,), jnp.float32),
        pltpu.VMEM((CH4 * 3,), jnp.float32),
        pltpu.VMEM((CH4 * 16,), jnp.float32),
    ],
)(_body)


def kernel(pos, edge_index):
    xcol = pos[:, 0]
    ycol = pos[:, 1]
    zcol = pos[:, 2]
    vec, sh, _ = _sc_call(xcol, ycol, zcol, edge_index[0], edge_index[1])
    return vec.reshape(N_EDGES, 3), sh.reshape(N_EDGES, 16)


# speedup vs baseline: 27.7511x; 4.3438x over previous
import functools
import math

import jax
import jax.numpy as jnp
from jax import lax
from jax.experimental import pallas as pl
from jax.experimental.pallas import tpu as pltpu
from jax.experimental.pallas import tpu_sc as plsc

N_NODES = 100000
N_EDGES = 3200000

NC, NS, L = 2, 16, 16
NW = NC * NS
BLK = 128
NBLK = N_EDGES // BLK
BPW = NBLK // NW
EXTRA = NBLK % NW
CB13 = 24
CB4 = 6
CH13 = CB13 * BLK
CH4 = CB4 * BLK

SQ3 = math.sqrt(3.0)
SQ5 = math.sqrt(5.0)
SQ7 = math.sqrt(7.0)
SQ15 = math.sqrt(15.0)
C3A = math.sqrt(42.0) / 6.0
C3B = math.sqrt(168.0) / 8.0
C3C = math.sqrt(7.0) / 2.0


def _rsqrt16(r2):
    i = plsc.bitcast(r2, jnp.int32)
    i = jnp.int32(0x5F3759DF) - lax.shift_right_logical(i, 1)
    y = plsc.bitcast(i, jnp.float32)
    for _ in range(3):
        y = y * (1.5 - 0.5 * r2 * y * y)
    return y


def _body(xcol, ycol, zcol, src, dst, vec_out, sh_out, comp, xtab,
          sbuf, dbuf, cbuf, vxb, vyb, vzb, vecb, shb):
    wid = lax.axis_index("s") * NC + lax.axis_index("c")
    nb = BPW + (wid < EXTRA).astype(jnp.int32)
    base_blk = wid * BPW + jnp.minimum(wid, EXTRA)

    for c, col in enumerate((xcol, ycol, zcol)):
        pltpu.sync_copy(col, xtab)

        def chunk13(k, _, c=c):
            blk0 = jnp.minimum(k * CB13, nb - CB13)
            e0 = pl.multiple_of((base_blk + blk0) * BLK, BLK)
            pltpu.sync_copy(src.at[pl.ds(e0, CH13)], sbuf)
            pltpu.sync_copy(dst.at[pl.ds(e0, CH13)], dbuf)

            def grp(g, _):
                sl = pl.ds(g * L, L)
                si = sbuf[sl]
                di = dbuf[sl]
                xs = plsc.load_gather(xtab, [si])
                xd = plsc.load_gather(xtab, [di])
                cbuf[sl] = xd - xs
                return 0

            lax.fori_loop(0, CH13 // L, grp, 0)
            pltpu.sync_copy(cbuf, comp.at[pl.ds(c * N_EDGES + e0, CH13)])
            return 0

        n13 = (nb + CB13 - 1) // CB13
        lax.fori_loop(0, n13, chunk13, 0)

    def chunk4(k, _):
        blk0 = jnp.minimum(k * CB4, nb - CB4)
        e0 = pl.multiple_of((base_blk + blk0) * BLK, BLK)
        pltpu.sync_copy(comp.at[pl.ds(e0, CH4)], vxb)
        pltpu.sync_copy(comp.at[pl.ds(N_EDGES + e0, CH4)], vyb)
        pltpu.sync_copy(comp.at[pl.ds(2 * N_EDGES + e0, CH4)], vzb)

        def grp(g, _):
            sl = pl.ds(g * L, L)
            x = vxb[sl]
            y = vyb[sl]
            z = vzb[sl]

            r2 = x * x + y * y + z * z
            rn = _rsqrt16(jnp.maximum(r2, 1e-24))
            ux = x * rn
            uy = y * rn
            uz = z * rn

            x2 = ux * ux
            y2 = uy * uy
            z2 = uz * uz
            x2z2 = x2 + z2
            sh20 = SQ15 * ux * uz
            sh21 = SQ15 * ux * uy
            sh22 = SQ5 * (y2 - 0.5 * x2z2)
            sh23 = SQ15 * uy * uz
            sh24 = (SQ15 / 2.0) * (z2 - x2)
            fy = 4.0 * y2 - x2z2
            sh = (
                jnp.full((L,), 1.0, jnp.float32),
                SQ3 * ux, SQ3 * uy, SQ3 * uz,
                sh20, sh21, sh22, sh23, sh24,
                C3A * (sh20 * uz + sh24 * ux),
                SQ7 * sh20 * uy,
                C3B * fy * ux,
                C3C * uy * (2.0 * y2 - 3.0 * x2z2),
                C3B * uz * fy,
                SQ7 * sh24 * uy,
                C3A * (sh24 * uz - sh20 * ux),
            )
            for ci, val in enumerate((x, y, z)):
                vecb[ci, sl] = val
            for k16, val in enumerate(sh):
                shb[k16, sl] = val
            return 0

        lax.fori_loop(0, CH4 // L, grp, 0)
        pltpu.sync_copy(vecb, vec_out.at[:, pl.ds(e0, CH4)])
        pltpu.sync_copy(shb, sh_out.at[:, pl.ds(e0, CH4)])
        return 0

    n4 = (nb + CB4 - 1) // CB4
    lax.fori_loop(0, n4, chunk4, 0)


_sc_call = functools.partial(
    pl.kernel,
    mesh=plsc.VectorSubcoreMesh(core_axis_name="c", subcore_axis_name="s"),
    compiler_params=pltpu.CompilerParams(needs_layout_passes=False),
    out_type=[
        jax.ShapeDtypeStruct((3, N_EDGES), jnp.float32),
        jax.ShapeDtypeStruct((16, N_EDGES), jnp.float32),
        jax.ShapeDtypeStruct((3 * N_EDGES,), jnp.float32),
    ],
    scratch_types=[
        pltpu.VMEM((N_NODES,), jnp.float32),
        pltpu.VMEM((CH13,), jnp.int32),
        pltpu.VMEM((CH13,), jnp.int32),
        pltpu.VMEM((CH13,), jnp.float32),
        pltpu.VMEM((CH4,), jnp.float32),
        pltpu.VMEM((CH4,), jnp.float32),
        pltpu.VMEM((CH4,), jnp.float32),
        pltpu.VMEM((3, CH4), jnp.float32),
        pltpu.VMEM((16, CH4), jnp.float32),
    ],
)(_body)


def kernel(pos, edge_index):
    vec3, sh16, _ = _sc_call(pos[:, 0], pos[:, 1], pos[:, 2],
                             edge_index[0], edge_index[1])
    return vec3.T, sh16.T


# speedup vs baseline: 55.1525x; 1.9874x over previous
import functools
import math

import jax
import jax.numpy as jnp
from jax import lax
from jax.experimental import pallas as pl
from jax.experimental.pallas import tpu as pltpu
from jax.experimental.pallas import tpu_sc as plsc

N_NODES = 100000
N_EDGES = 3200000

NC, NS, L = 2, 16, 16
NW = NC * NS
BLK = 128
NBLK = N_EDGES // BLK
BPW = NBLK // NW
EXTRA = NBLK % NW
BPW_MAX = BPW + (1 if EXTRA else 0)
CB13 = 12
CB4 = 4
CH13 = CB13 * BLK
CH4 = CB4 * BLK
N13 = -(-BPW_MAX // CB13)
N13 += N13 % 2
N4 = -(-BPW_MAX // CB4)
N4 += N4 % 2

SQ3 = math.sqrt(3.0)
SQ5 = math.sqrt(5.0)
SQ7 = math.sqrt(7.0)
SQ15 = math.sqrt(15.0)
C3A = math.sqrt(42.0) / 6.0
C3B = math.sqrt(168.0) / 8.0
C3C = math.sqrt(7.0) / 2.0


def _rsqrt16(r2):
    i = plsc.bitcast(r2, jnp.int32)
    i = jnp.int32(0x5F3759DF) - lax.shift_right_logical(i, 1)
    y = plsc.bitcast(i, jnp.float32)
    for _ in range(3):
        y = y * (1.5 - 0.5 * r2 * y * y)
    return y


def _body(xcol, ycol, zcol, src, dst, vec_out, sh_out, comp, xtab,
          sbuf0, sbuf1, dbuf0, dbuf1, cbuf0, cbuf1,
          vecb0, vecb1, shb0, shb1, isem0, isem1, osem0, osem1):
    wid = lax.axis_index("s") * NC + lax.axis_index("c")
    nb = BPW + (wid < EXTRA).astype(jnp.int32)
    base_blk = wid * BPW + jnp.minimum(wid, EXTRA)

    sbufs = (sbuf0, sbuf1)
    dbufs = (dbuf0, dbuf1)
    cbufs = (cbuf0, cbuf1)
    vecbs = (vecb0, vecb1)
    shbs = (shb0, shb1)
    isems = (isem0, isem1)
    osems = (osem0, osem1)

    def e0_of(k, cb):
        blk0 = jnp.minimum(k * cb, nb - cb)
        return pl.multiple_of((base_blk + blk0) * BLK, BLK)

    for c, col in enumerate((xcol, ycol, zcol)):
        pltpu.sync_copy(col, xtab)

        for b in (0, 1):
            e0 = e0_of(b, CB13)
            pltpu.async_copy(src.at[pl.ds(e0, CH13)], sbufs[b], isems[b])
            pltpu.async_copy(dst.at[pl.ds(e0, CH13)], dbufs[b], isems[b])

        def pair13(j, _, c=c):
            for b in (0, 1):
                k = 2 * j + b
                e0 = e0_of(k, CB13)
                pltpu.make_async_copy(
                    src.at[pl.ds(e0, CH13)], sbufs[b], isems[b]).wait()
                pltpu.make_async_copy(
                    dst.at[pl.ds(e0, CH13)], dbufs[b], isems[b]).wait()

                @pl.when(k >= 2)
                def _(b=b, e0=e0):
                    pltpu.make_async_copy(
                        cbufs[b], comp.at[pl.ds(c * N_EDGES + e0, CH13)],
                        osems[b]).wait()

                def grp(g, _, b=b):
                    for u in range(4):
                        sl = pl.ds(g * (4 * L) + u * L, L)
                        si = sbufs[b][sl]
                        di = dbufs[b][sl]
                        xs = plsc.load_gather(xtab, [si])
                        xd = plsc.load_gather(xtab, [di])
                        cbufs[b][sl] = xd - xs
                    return 0

                lax.fori_loop(0, CH13 // (4 * L), grp, 0)
                pltpu.async_copy(
                    cbufs[b], comp.at[pl.ds(c * N_EDGES + e0, CH13)], osems[b])

                @pl.when(k + 2 < N13)
                def _(b=b, k=k):
                    e2 = e0_of(k + 2, CB13)
                    pltpu.async_copy(src.at[pl.ds(e2, CH13)], sbufs[b], isems[b])
                    pltpu.async_copy(dst.at[pl.ds(e2, CH13)], dbufs[b], isems[b])
            return 0

        lax.fori_loop(0, N13 // 2, pair13, 0)
        for b in (0, 1):
            e0 = e0_of(N13 - 2 + b, CB13)
            pltpu.make_async_copy(
                cbufs[b], comp.at[pl.ds(c * N_EDGES + e0, CH13)],
                osems[b]).wait()

    def start_in4(k, b):
        e0 = e0_of(k, CB4)
        for c in range(3):
            pltpu.async_copy(comp.at[pl.ds(c * N_EDGES + e0, CH4)],
                             cbufs[b].at[pl.ds(c * CH4, CH4)], isems[b])

    def wait_in4(k, b):
        e0 = e0_of(k, CB4)
        for c in range(3):
            pltpu.make_async_copy(comp.at[pl.ds(c * N_EDGES + e0, CH4)],
                                  cbufs[b].at[pl.ds(c * CH4, CH4)],
                                  isems[b]).wait()

    def wait_out4(k, b):
        e0 = e0_of(k, CB4)
        pltpu.make_async_copy(
            vecbs[b], vec_out.at[:, pl.ds(e0, CH4)], osems[b]).wait()
        pltpu.make_async_copy(
            shbs[b], sh_out.at[:, pl.ds(e0, CH4)], osems[b]).wait()

    for b in (0, 1):
        start_in4(b, b)

    def pair4(j, _):
        for b in (0, 1):
            k = 2 * j + b
            e0 = e0_of(k, CB4)
            wait_in4(k, b)

            @pl.when(k >= 2)
            def _(k=k, b=b):
                wait_out4(k, b)

            def grp(g, _, b=b):
                for u in range(2):
                    o = g * (2 * L) + u * L
                    x = cbufs[b][pl.ds(o, L)]
                    y = cbufs[b][pl.ds(CH4 + o, L)]
                    z = cbufs[b][pl.ds(2 * CH4 + o, L)]

                    r2 = x * x + y * y + z * z
                    rn = _rsqrt16(jnp.maximum(r2, 1e-24))
                    ux = x * rn
                    uy = y * rn
                    uz = z * rn

                    x2 = ux * ux
                    y2 = uy * uy
                    z2 = uz * uz
                    x2z2 = x2 + z2
                    sh20 = SQ15 * ux * uz
                    sh21 = SQ15 * ux * uy
                    sh22 = SQ5 * (y2 - 0.5 * x2z2)
                    sh23 = SQ15 * uy * uz
                    sh24 = (SQ15 / 2.0) * (z2 - x2)
                    fy = 4.0 * y2 - x2z2
                    sh = (
                        jnp.full((L,), 1.0, jnp.float32),
                        SQ3 * ux, SQ3 * uy, SQ3 * uz,
                        sh20, sh21, sh22, sh23, sh24,
                        C3A * (sh20 * uz + sh24 * ux),
                        SQ7 * sh20 * uy,
                        C3B * fy * ux,
                        C3C * uy * (2.0 * y2 - 3.0 * x2z2),
                        C3B * uz * fy,
                        SQ7 * sh24 * uy,
                        C3A * (sh24 * uz - sh20 * ux),
                    )
                    sl = pl.ds(o, L)
                    for ci, val in enumerate((x, y, z)):
                        vecbs[b][ci, sl] = val
                    for k16, val in enumerate(sh):
                        shbs[b][k16, sl] = val
                return 0

            lax.fori_loop(0, CH4 // (2 * L), grp, 0)
            pltpu.async_copy(vecbs[b], vec_out.at[:, pl.ds(e0, CH4)], osems[b])
            pltpu.async_copy(shbs[b], sh_out.at[:, pl.ds(e0, CH4)], osems[b])

            @pl.when(k + 2 < N4)
            def _(k=k, b=b):
                start_in4(k + 2, b)
        return 0

    lax.fori_loop(0, N4 // 2, pair4, 0)
    for b in (0, 1):
        wait_out4(N4 - 2 + b, b)


_sc_call = functools.partial(
    pl.kernel,
    mesh=plsc.VectorSubcoreMesh(core_axis_name="c", subcore_axis_name="s"),
    compiler_params=pltpu.CompilerParams(needs_layout_passes=False),
    out_type=[
        jax.ShapeDtypeStruct((3, N_EDGES), jnp.float32),
        jax.ShapeDtypeStruct((16, N_EDGES), jnp.float32),
        jax.ShapeDtypeStruct((3 * N_EDGES,), jnp.float32),
    ],
    scratch_types=[
        pltpu.VMEM((N_NODES,), jnp.float32),
        pltpu.VMEM((CH13,), jnp.int32),
        pltpu.VMEM((CH13,), jnp.int32),
        pltpu.VMEM((CH13,), jnp.int32),
        pltpu.VMEM((CH13,), jnp.int32),
        pltpu.VMEM((CH13,), jnp.float32),
        pltpu.VMEM((CH13,), jnp.float32),
        pltpu.VMEM((3, CH4), jnp.float32),
        pltpu.VMEM((3, CH4), jnp.float32),
        pltpu.VMEM((16, CH4), jnp.float32),
        pltpu.VMEM((16, CH4), jnp.float32),
        pltpu.SemaphoreType.DMA,
        pltpu.SemaphoreType.DMA,
        pltpu.SemaphoreType.DMA,
        pltpu.SemaphoreType.DMA,
    ],
)(_body)


def kernel(pos, edge_index):
    vec3, sh16, _ = _sc_call(pos[:, 0], pos[:, 1], pos[:, 2],
                             edge_index[0], edge_index[1])
    return vec3.T, sh16.T


# speedup vs baseline: 56.1409x; 1.0179x over previous
import functools
import math

import jax
import jax.numpy as jnp
from jax import lax
from jax.experimental import pallas as pl
from jax.experimental.pallas import tpu as pltpu
from jax.experimental.pallas import tpu_sc as plsc

N_NODES = 100000
N_EDGES = 3200000

NC, NS, L = 2, 16, 16
NW = NC * NS
BLK = 128
NBLK = N_EDGES // BLK
BPW = NBLK // NW
EXTRA = NBLK % NW
BPW_MAX = BPW + (1 if EXTRA else 0)
CB13 = 12
CB4 = 4
CH13 = CB13 * BLK
CH4 = CB4 * BLK
N13 = -(-BPW_MAX // CB13)
N13 += N13 % 2
N4 = -(-BPW_MAX // CB4)
N4 += N4 % 2

SQ3 = math.sqrt(3.0)
SQ5 = math.sqrt(5.0)
SQ7 = math.sqrt(7.0)
SQ15 = math.sqrt(15.0)
C3A = math.sqrt(42.0) / 6.0
C3B = math.sqrt(168.0) / 8.0
C3C = math.sqrt(7.0) / 2.0


def _rsqrt16(r2):
    i = plsc.bitcast(r2, jnp.int32)
    i = jnp.int32(0x5F3759DF) - lax.shift_right_logical(i, 1)
    y = plsc.bitcast(i, jnp.float32)
    for _ in range(3):
        y = y * (1.5 - 0.5 * r2 * y * y)
    return y


def _body(xcol, ycol, zcol, src, dst, vec_out, sh_out, comp, xtab,
          sbuf0, sbuf1, dbuf0, dbuf1, cbuf0, cbuf1,
          vecb0, vecb1, shb0, shb1, isem0, isem1, osem0, osem1):
    wid = lax.axis_index("s") * NC + lax.axis_index("c")
    nb = BPW + (wid < EXTRA).astype(jnp.int32)
    base_blk = wid * BPW + jnp.minimum(wid, EXTRA)

    sbufs = (sbuf0, sbuf1)
    dbufs = (dbuf0, dbuf1)
    cbufs = (cbuf0, cbuf1)
    vecbs = (vecb0, vecb1)
    shbs = (shb0, shb1)
    isems = (isem0, isem1)
    osems = (osem0, osem1)

    def e0_of(k, cb):
        blk0 = jnp.minimum(k * cb, nb - cb)
        return pl.multiple_of((base_blk + blk0) * BLK, BLK)

    for c, col in enumerate((xcol, ycol)):
        pltpu.sync_copy(col, xtab)

        for b in (0, 1):
            e0 = e0_of(b, CB13)
            pltpu.async_copy(src.at[pl.ds(e0, CH13)], sbufs[b], isems[b])
            pltpu.async_copy(dst.at[pl.ds(e0, CH13)], dbufs[b], isems[b])

        def pair13(j, _, c=c):
            for b in (0, 1):
                k = 2 * j + b
                e0 = e0_of(k, CB13)
                pltpu.make_async_copy(
                    src.at[pl.ds(e0, CH13)], sbufs[b], isems[b]).wait()
                pltpu.make_async_copy(
                    dst.at[pl.ds(e0, CH13)], dbufs[b], isems[b]).wait()

                @pl.when(k >= 2)
                def _(b=b, e0=e0):
                    pltpu.make_async_copy(
                        cbufs[b], comp.at[pl.ds(c * N_EDGES + e0, CH13)],
                        osems[b]).wait()

                def grp(g, _, b=b):
                    for u in range(4):
                        sl = pl.ds(g * (4 * L) + u * L, L)
                        si = sbufs[b][sl]
                        di = dbufs[b][sl]
                        xs = plsc.load_gather(xtab, [si])
                        xd = plsc.load_gather(xtab, [di])
                        cbufs[b][sl] = xd - xs
                    return 0

                lax.fori_loop(0, CH13 // (4 * L), grp, 0)
                pltpu.async_copy(
                    cbufs[b], comp.at[pl.ds(c * N_EDGES + e0, CH13)], osems[b])

                @pl.when(k + 2 < N13)
                def _(b=b, k=k):
                    e2 = e0_of(k + 2, CB13)
                    pltpu.async_copy(src.at[pl.ds(e2, CH13)], sbufs[b], isems[b])
                    pltpu.async_copy(dst.at[pl.ds(e2, CH13)], dbufs[b], isems[b])
            return 0

        lax.fori_loop(0, N13 // 2, pair13, 0)
        for b in (0, 1):
            e0 = e0_of(N13 - 2 + b, CB13)
            pltpu.make_async_copy(
                cbufs[b], comp.at[pl.ds(c * N_EDGES + e0, CH13)],
                osems[b]).wait()

    pltpu.sync_copy(zcol, xtab)
    ones = jnp.full((L,), 1.0, jnp.float32)
    for b in (0, 1):
        for g in range(CH4 // L):
            shbs[b][0, pl.ds(g * L, L)] = ones

    def start_in4(k, b):
        e0 = e0_of(k, CB4)
        pltpu.async_copy(src.at[pl.ds(e0, CH4)],
                         sbufs[b].at[pl.ds(0, CH4)], isems[b])
        pltpu.async_copy(dst.at[pl.ds(e0, CH4)],
                         dbufs[b].at[pl.ds(0, CH4)], isems[b])
        pltpu.async_copy(comp.at[pl.ds(e0, CH4)],
                         vecbs[b].at[0], isems[b])
        pltpu.async_copy(comp.at[pl.ds(N_EDGES + e0, CH4)],
                         vecbs[b].at[1], isems[b])

    def wait_in4(k, b):
        e0 = e0_of(k, CB4)
        pltpu.make_async_copy(src.at[pl.ds(e0, CH4)],
                              sbufs[b].at[pl.ds(0, CH4)], isems[b]).wait()
        pltpu.make_async_copy(dst.at[pl.ds(e0, CH4)],
                              dbufs[b].at[pl.ds(0, CH4)], isems[b]).wait()
        pltpu.make_async_copy(comp.at[pl.ds(e0, CH4)],
                              vecbs[b].at[0], isems[b]).wait()
        pltpu.make_async_copy(comp.at[pl.ds(N_EDGES + e0, CH4)],
                              vecbs[b].at[1], isems[b]).wait()

    def wait_out4(k, b):
        e0 = e0_of(k, CB4)
        pltpu.make_async_copy(
            vecbs[b], vec_out.at[:, pl.ds(e0, CH4)], osems[b]).wait()
        pltpu.make_async_copy(
            shbs[b], sh_out.at[:, pl.ds(e0, CH4)], osems[b]).wait()

    for b in (0, 1):
        start_in4(b, b)

    def pair4(j, _):
        for b in (0, 1):
            k = 2 * j + b
            e0 = e0_of(k, CB4)
            wait_in4(k, b)

            @pl.when(k >= 2)
            def _(k=k, b=b):
                wait_out4(k, b)

            def grp(g, _, b=b):
                for u in range(2):
                    o = g * (2 * L) + u * L
                    sl = pl.ds(o, L)
                    si = sbufs[b][sl]
                    di = dbufs[b][sl]
                    zs = plsc.load_gather(xtab, [si])
                    zd = plsc.load_gather(xtab, [di])
                    z = zd - zs
                    x = vecbs[b][0, sl]
                    y = vecbs[b][1, sl]
                    vecbs[b][2, sl] = z

                    r2 = x * x + y * y + z * z
                    rn = _rsqrt16(jnp.maximum(r2, 1e-24))
                    ux = x * rn
                    uy = y * rn
                    uz = z * rn

                    x2 = ux * ux
                    y2 = uy * uy
                    z2 = uz * uz
                    x2z2 = x2 + z2
                    sh20 = SQ15 * ux * uz
                    sh21 = SQ15 * ux * uy
                    sh22 = SQ5 * (y2 - 0.5 * x2z2)
                    sh23 = SQ15 * uy * uz
                    sh24 = (SQ15 / 2.0) * (z2 - x2)
                    fy = 4.0 * y2 - x2z2
                    sh = (
                        SQ3 * ux, SQ3 * uy, SQ3 * uz,
                        sh20, sh21, sh22, sh23, sh24,
                        C3A * (sh20 * uz + sh24 * ux),
                        SQ7 * sh20 * uy,
                        C3B * fy * ux,
                        C3C * uy * (2.0 * y2 - 3.0 * x2z2),
                        C3B * uz * fy,
                        SQ7 * sh24 * uy,
                        C3A * (sh24 * uz - sh20 * ux),
                    )
                    for k16, val in enumerate(sh):
                        shbs[b][k16 + 1, sl] = val
                return 0

            lax.fori_loop(0, CH4 // (2 * L), grp, 0)
            pltpu.async_copy(vecbs[b], vec_out.at[:, pl.ds(e0, CH4)], osems[b])
            pltpu.async_copy(shbs[b], sh_out.at[:, pl.ds(e0, CH4)], osems[b])

            @pl.when(k + 2 < N4)
            def _(k=k, b=b):
                start_in4(k + 2, b)
        return 0

    lax.fori_loop(0, N4 // 2, pair4, 0)
    for b in (0, 1):
        wait_out4(N4 - 2 + b, b)


_sc_call = functools.partial(
    pl.kernel,
    mesh=plsc.VectorSubcoreMesh(core_axis_name="c", subcore_axis_name="s"),
    compiler_params=pltpu.CompilerParams(needs_layout_passes=False),
    out_type=[
        jax.ShapeDtypeStruct((3, N_EDGES), jnp.float32),
        jax.ShapeDtypeStruct((16, N_EDGES), jnp.float32),
        jax.ShapeDtypeStruct((2 * N_EDGES,), jnp.float32),
    ],
    scratch_types=[
        pltpu.VMEM((N_NODES,), jnp.float32),
        pltpu.VMEM((CH13,), jnp.int32),
        pltpu.VMEM((CH13,), jnp.int32),
        pltpu.VMEM((CH13,), jnp.int32),
        pltpu.VMEM((CH13,), jnp.int32),
        pltpu.VMEM((CH13,), jnp.float32),
        pltpu.VMEM((CH13,), jnp.float32),
        pltpu.VMEM((3, CH4), jnp.float32),
        pltpu.VMEM((3, CH4), jnp.float32),
        pltpu.VMEM((16, CH4), jnp.float32),
        pltpu.VMEM((16, CH4), jnp.float32),
        pltpu.SemaphoreType.DMA,
        pltpu.SemaphoreType.DMA,
        pltpu.SemaphoreType.DMA,
        pltpu.SemaphoreType.DMA,
    ],
)(_body)


def kernel(pos, edge_index):
    vec3, sh16, _ = _sc_call(pos[:, 0], pos[:, 1], pos[:, 2],
                             edge_index[0], edge_index[1])
    return vec3.T, sh16.T


# speedup vs baseline: 77.9984x; 1.3893x over previous
import functools
import math

import jax
import jax.numpy as jnp
from jax import lax
from jax.experimental import pallas as pl
from jax.experimental.pallas import tpu as pltpu
from jax.experimental.pallas import tpu_sc as plsc

N_NODES = 100000
N_EDGES = 3200000

NC, NS, L = 2, 16, 16
NW = NC * NS
BLK = 128
NBLK = N_EDGES // BLK
BPW = NBLK // NW
EXTRA = NBLK % NW
BPW_MAX = BPW + (1 if EXTRA else 0)
CB13 = 12
CB4 = 4
CH13 = CB13 * BLK
CH4 = CB4 * BLK
N13 = -(-BPW_MAX // CB13)
N13 += N13 % 2
N4 = -(-BPW_MAX // CB4)
N4 += N4 % 2

SQ3 = math.sqrt(3.0)
SQ5 = math.sqrt(5.0)
SQ7 = math.sqrt(7.0)
SQ15 = math.sqrt(15.0)
C3A = math.sqrt(42.0) / 6.0
C3B = math.sqrt(168.0) / 8.0
C3C = math.sqrt(7.0) / 2.0


def _rsqrt16(r2):
    i = plsc.bitcast(r2, jnp.int32)
    i = jnp.int32(0x5F3759DF) - lax.shift_right_logical(i, 1)
    y = plsc.bitcast(i, jnp.float32)
    for _ in range(3):
        y = y * (1.5 - 0.5 * r2 * y * y)
    return y


def _body(xcol, ycol, zcol, src, dst, vec_out, sh_out, comp, xtab,
          sbuf0, sbuf1, dbuf0, dbuf1, cbuf0, cbuf1,
          vecb0, vecb1, shb0, shb1, isem0, isem1, osem0, osem1):
    wid = lax.axis_index("s") * NC + lax.axis_index("c")
    nb = BPW + (wid < EXTRA).astype(jnp.int32)
    base_blk = wid * BPW + jnp.minimum(wid, EXTRA)

    sbufs = (sbuf0, sbuf1)
    dbufs = (dbuf0, dbuf1)
    cbufs = (cbuf0, cbuf1)
    vecbs = (vecb0, vecb1)
    shbs = (shb0, shb1)
    isems = (isem0, isem1)
    osems = (osem0, osem1)

    def e0_of(k, cb):
        blk0 = jnp.minimum(k * cb, nb - cb)
        return pl.multiple_of((base_blk + blk0) * BLK, BLK)

    for c, col in enumerate((xcol, ycol)):
        pltpu.sync_copy(col, xtab)

        for b in (0, 1):
            e0 = e0_of(b, CB13)
            pltpu.async_copy(src.at[pl.ds(e0, CH13)], sbufs[b], isems[b])
            pltpu.async_copy(dst.at[pl.ds(e0, CH13)], dbufs[b], isems[b])

        def pair13(j, _, c=c):
            for b in (0, 1):
                k = 2 * j + b
                e0 = e0_of(k, CB13)
                pltpu.make_async_copy(
                    src.at[pl.ds(e0, CH13)], sbufs[b], isems[b]).wait()
                pltpu.make_async_copy(
                    dst.at[pl.ds(e0, CH13)], dbufs[b], isems[b]).wait()

                @pl.when(k >= 2)
                def _(b=b, e0=e0):
                    pltpu.make_async_copy(
                        cbufs[b], comp.at[pl.ds(c * N_EDGES + e0, CH13)],
                        osems[b]).wait()

                @plsc.parallel_loop(0, CH13 // L, 1, unroll=4)
                def _(g, b=b):
                    sl = pl.ds(g * L, L)
                    si = sbufs[b][sl]
                    di = dbufs[b][sl]
                    xs = plsc.load_gather(xtab, [si])
                    xd = plsc.load_gather(xtab, [di])
                    cbufs[b][sl] = xd - xs
                pltpu.async_copy(
                    cbufs[b], comp.at[pl.ds(c * N_EDGES + e0, CH13)], osems[b])

                @pl.when(k + 2 < N13)
                def _(b=b, k=k):
                    e2 = e0_of(k + 2, CB13)
                    pltpu.async_copy(src.at[pl.ds(e2, CH13)], sbufs[b], isems[b])
                    pltpu.async_copy(dst.at[pl.ds(e2, CH13)], dbufs[b], isems[b])
            return 0

        lax.fori_loop(0, N13 // 2, pair13, 0)
        for b in (0, 1):
            e0 = e0_of(N13 - 2 + b, CB13)
            pltpu.make_async_copy(
                cbufs[b], comp.at[pl.ds(c * N_EDGES + e0, CH13)],
                osems[b]).wait()

    pltpu.sync_copy(zcol, xtab)
    ones = jnp.full((L,), 1.0, jnp.float32)
    for b in (0, 1):
        for g in range(CH4 // L):
            shbs[b][0, pl.ds(g * L, L)] = ones

    def start_in4(k, b):
        e0 = e0_of(k, CB4)
        pltpu.async_copy(src.at[pl.ds(e0, CH4)],
                         sbufs[b].at[pl.ds(0, CH4)], isems[b])
        pltpu.async_copy(dst.at[pl.ds(e0, CH4)],
                         dbufs[b].at[pl.ds(0, CH4)], isems[b])
        pltpu.async_copy(comp.at[pl.ds(e0, CH4)],
                         vecbs[b].at[0], isems[b])
        pltpu.async_copy(comp.at[pl.ds(N_EDGES + e0, CH4)],
                         vecbs[b].at[1], isems[b])

    def wait_in4(k, b):
        e0 = e0_of(k, CB4)
        pltpu.make_async_copy(src.at[pl.ds(e0, CH4)],
                              sbufs[b].at[pl.ds(0, CH4)], isems[b]).wait()
        pltpu.make_async_copy(dst.at[pl.ds(e0, CH4)],
                              dbufs[b].at[pl.ds(0, CH4)], isems[b]).wait()
        pltpu.make_async_copy(comp.at[pl.ds(e0, CH4)],
                              vecbs[b].at[0], isems[b]).wait()
        pltpu.make_async_copy(comp.at[pl.ds(N_EDGES + e0, CH4)],
                              vecbs[b].at[1], isems[b]).wait()

    def wait_out4(k, b):
        e0 = e0_of(k, CB4)
        pltpu.make_async_copy(
            vecbs[b], vec_out.at[:, pl.ds(e0, CH4)], osems[b]).wait()
        pltpu.make_async_copy(
            shbs[b], sh_out.at[:, pl.ds(e0, CH4)], osems[b]).wait()

    for b in (0, 1):
        start_in4(b, b)

    def pair4(j, _):
        for b in (0, 1):
            k = 2 * j + b
            e0 = e0_of(k, CB4)
            wait_in4(k, b)

            @pl.when(k >= 2)
            def _(k=k, b=b):
                wait_out4(k, b)

            @plsc.parallel_loop(0, CH4 // L, 1, unroll=2)
            def _(g, b=b):
                sl = pl.ds(g * L, L)
                si = sbufs[b][sl]
                di = dbufs[b][sl]
                zs = plsc.load_gather(xtab, [si])
                zd = plsc.load_gather(xtab, [di])
                z = zd - zs
                x = vecbs[b][0, sl]
                y = vecbs[b][1, sl]
                vecbs[b][2, sl] = z

                r2 = x * x + y * y + z * z
                rn = _rsqrt16(jnp.maximum(r2, 1e-24))
                ux = x * rn
                uy = y * rn
                uz = z * rn
                shb = shbs[b]
                shb[1, sl] = SQ3 * ux
                shb[2, sl] = SQ3 * uy
                shb[3, sl] = SQ3 * uz

                x2 = ux * ux
                y2 = uy * uy
                z2 = uz * uz
                x2z2 = x2 + z2
                sh20 = SQ15 * ux * uz
                sh24 = (SQ15 / 2.0) * (z2 - x2)
                shb[4, sl] = sh20
                shb[5, sl] = SQ15 * ux * uy
                shb[6, sl] = SQ5 * (y2 - 0.5 * x2z2)
                shb[7, sl] = SQ15 * uy * uz
                shb[8, sl] = sh24
                fy = 4.0 * y2 - x2z2
                shb[9, sl] = C3A * (sh20 * uz + sh24 * ux)
                shb[10, sl] = SQ7 * sh20 * uy
                shb[11, sl] = C3B * fy * ux
                shb[12, sl] = C3C * uy * (2.0 * y2 - 3.0 * x2z2)
                shb[13, sl] = C3B * uz * fy
                shb[14, sl] = SQ7 * sh24 * uy
                shb[15, sl] = C3A * (sh24 * uz - sh20 * ux)
            pltpu.async_copy(vecbs[b], vec_out.at[:, pl.ds(e0, CH4)], osems[b])
            pltpu.async_copy(shbs[b], sh_out.at[:, pl.ds(e0, CH4)], osems[b])

            @pl.when(k + 2 < N4)
            def _(k=k, b=b):
                start_in4(k + 2, b)
        return 0

    lax.fori_loop(0, N4 // 2, pair4, 0)
    for b in (0, 1):
        wait_out4(N4 - 2 + b, b)


_sc_call = functools.partial(
    pl.kernel,
    mesh=plsc.VectorSubcoreMesh(core_axis_name="c", subcore_axis_name="s"),
    compiler_params=pltpu.CompilerParams(needs_layout_passes=False),
    out_type=[
        jax.ShapeDtypeStruct((3, N_EDGES), jnp.float32),
        jax.ShapeDtypeStruct((16, N_EDGES), jnp.float32),
        jax.ShapeDtypeStruct((2 * N_EDGES,), jnp.float32),
    ],
    scratch_types=[
        pltpu.VMEM((N_NODES,), jnp.float32),
        pltpu.VMEM((CH13,), jnp.int32),
        pltpu.VMEM((CH13,), jnp.int32),
        pltpu.VMEM((CH13,), jnp.int32),
        pltpu.VMEM((CH13,), jnp.int32),
        pltpu.VMEM((CH13,), jnp.float32),
        pltpu.VMEM((CH13,), jnp.float32),
        pltpu.VMEM((3, CH4), jnp.float32),
        pltpu.VMEM((3, CH4), jnp.float32),
        pltpu.VMEM((16, CH4), jnp.float32),
        pltpu.VMEM((16, CH4), jnp.float32),
        pltpu.SemaphoreType.DMA,
        pltpu.SemaphoreType.DMA,
        pltpu.SemaphoreType.DMA,
        pltpu.SemaphoreType.DMA,
    ],
)(_body)


def kernel(pos, edge_index):
    vec3, sh16, _ = _sc_call(pos[:, 0], pos[:, 1], pos[:, 2],
                             edge_index[0], edge_index[1])
    return vec3.T, sh16.T


# speedup vs baseline: 80.0987x; 1.0269x over previous
import functools
import math

import jax
import jax.numpy as jnp
from jax import lax
from jax.experimental import pallas as pl
from jax.experimental.pallas import tpu as pltpu
from jax.experimental.pallas import tpu_sc as plsc

N_NODES = 100000
N_EDGES = 3200000

NC, NS, L = 2, 16, 16
NW = NC * NS
BLK = 128
NBLK = N_EDGES // BLK
BPW = NBLK // NW
EXTRA = NBLK % NW
BPW_MAX = BPW + (1 if EXTRA else 0)
CB13 = 12
CB4 = 4
CH13 = CB13 * BLK
CH4 = CB4 * BLK
N13 = -(-BPW_MAX // CB13)
N13 += N13 % 2
N4 = -(-BPW_MAX // CB4)
N4 += N4 % 2

SQ3 = math.sqrt(3.0)
SQ5 = math.sqrt(5.0)
SQ7 = math.sqrt(7.0)
SQ15 = math.sqrt(15.0)
C3A = math.sqrt(42.0) / 6.0
C3B = math.sqrt(168.0) / 8.0
C3C = math.sqrt(7.0) / 2.0


def _rsqrt16(r2):
    i = plsc.bitcast(r2, jnp.int32)
    i = jnp.int32(0x5F3759DF) - lax.shift_right_logical(i, 1)
    y = plsc.bitcast(i, jnp.float32)
    for _ in range(1):
        y = y * (1.5 - 0.5 * r2 * y * y)
    return y


def _body(xcol, ycol, zcol, src, dst, vec_out, sh_out, comp, xtab,
          sbuf0, sbuf1, dbuf0, dbuf1, cbuf0, cbuf1,
          vecb0, vecb1, shb0, shb1, isem0, isem1, osem0, osem1):
    wid = lax.axis_index("s") * NC + lax.axis_index("c")
    nb = BPW + (wid < EXTRA).astype(jnp.int32)
    base_blk = wid * BPW + jnp.minimum(wid, EXTRA)

    sbufs = (sbuf0, sbuf1)
    dbufs = (dbuf0, dbuf1)
    cbufs = (cbuf0, cbuf1)
    vecbs = (vecb0, vecb1)
    shbs = (shb0, shb1)
    isems = (isem0, isem1)
    osems = (osem0, osem1)

    def e0_of(k, cb):
        blk0 = jnp.minimum(k * cb, nb - cb)
        return pl.multiple_of((base_blk + blk0) * BLK, BLK)

    for c, col in enumerate((xcol, ycol)):
        pltpu.sync_copy(col, xtab)

        for b in (0, 1):
            e0 = e0_of(b, CB13)
            pltpu.async_copy(src.at[pl.ds(e0, CH13)], sbufs[b], isems[b])
            pltpu.async_copy(dst.at[pl.ds(e0, CH13)], dbufs[b], isems[b])

        def pair13(j, _, c=c):
            for b in (0, 1):
                k = 2 * j + b
                e0 = e0_of(k, CB13)
                pltpu.make_async_copy(
                    src.at[pl.ds(e0, CH13)], sbufs[b], isems[b]).wait()
                pltpu.make_async_copy(
                    dst.at[pl.ds(e0, CH13)], dbufs[b], isems[b]).wait()

                @pl.when(k >= 2)
                def _(b=b, e0=e0):
                    pltpu.make_async_copy(
                        cbufs[b], comp.at[pl.ds(c * N_EDGES + e0, CH13)],
                        osems[b]).wait()

                @plsc.parallel_loop(0, CH13 // L, 1, unroll=4)
                def _(g, b=b):
                    sl = pl.ds(g * L, L)
                    si = sbufs[b][sl]
                    di = dbufs[b][sl]
                    xs = plsc.load_gather(xtab, [si])
                    xd = plsc.load_gather(xtab, [di])
                    cbufs[b][sl] = xd - xs
                pltpu.async_copy(
                    cbufs[b], comp.at[pl.ds(c * N_EDGES + e0, CH13)], osems[b])

                @pl.when(k + 2 < N13)
                def _(b=b, k=k):
                    e2 = e0_of(k + 2, CB13)
                    pltpu.async_copy(src.at[pl.ds(e2, CH13)], sbufs[b], isems[b])
                    pltpu.async_copy(dst.at[pl.ds(e2, CH13)], dbufs[b], isems[b])
            return 0

        lax.fori_loop(0, N13 // 2, pair13, 0)
        for b in (0, 1):
            e0 = e0_of(N13 - 2 + b, CB13)
            pltpu.make_async_copy(
                cbufs[b], comp.at[pl.ds(c * N_EDGES + e0, CH13)],
                osems[b]).wait()

    pltpu.sync_copy(zcol, xtab)
    ones = jnp.full((L,), 1.0, jnp.float32)
    for b in (0, 1):
        for g in range(CH4 // L):
            shbs[b][0, pl.ds(g * L, L)] = ones

    def start_in4(k, b):
        e0 = e0_of(k, CB4)
        pltpu.async_copy(src.at[pl.ds(e0, CH4)],
                         sbufs[b].at[pl.ds(0, CH4)], isems[b])
        pltpu.async_copy(dst.at[pl.ds(e0, CH4)],
                         dbufs[b].at[pl.ds(0, CH4)], isems[b])
        pltpu.async_copy(comp.at[pl.ds(e0, CH4)],
                         vecbs[b].at[0], isems[b])
        pltpu.async_copy(comp.at[pl.ds(N_EDGES + e0, CH4)],
                         vecbs[b].at[1], isems[b])

    def wait_in4(k, b):
        e0 = e0_of(k, CB4)
        pltpu.make_async_copy(src.at[pl.ds(e0, CH4)],
                              sbufs[b].at[pl.ds(0, CH4)], isems[b]).wait()
        pltpu.make_async_copy(dst.at[pl.ds(e0, CH4)],
                              dbufs[b].at[pl.ds(0, CH4)], isems[b]).wait()
        pltpu.make_async_copy(comp.at[pl.ds(e0, CH4)],
                              vecbs[b].at[0], isems[b]).wait()
        pltpu.make_async_copy(comp.at[pl.ds(N_EDGES + e0, CH4)],
                              vecbs[b].at[1], isems[b]).wait()

    def wait_out4(k, b):
        e0 = e0_of(k, CB4)
        pltpu.make_async_copy(
            vecbs[b], vec_out.at[:, pl.ds(e0, CH4)], osems[b]).wait()
        pltpu.make_async_copy(
            shbs[b], sh_out.at[:, pl.ds(e0, CH4)], osems[b]).wait()

    for b in (0, 1):
        start_in4(b, b)

    def pair4(j, _):
        for b in (0, 1):
            k = 2 * j + b
            e0 = e0_of(k, CB4)
            wait_in4(k, b)

            @pl.when(k >= 2)
            def _(k=k, b=b):
                wait_out4(k, b)

            @plsc.parallel_loop(0, CH4 // L, 1, unroll=4)
            def _(g, b=b):
                sl = pl.ds(g * L, L)
                si = sbufs[b][sl]
                di = dbufs[b][sl]
                zs = plsc.load_gather(xtab, [si])
                zd = plsc.load_gather(xtab, [di])
                z = zd - zs
                x = vecbs[b][0, sl]
                y = vecbs[b][1, sl]
                vecbs[b][2, sl] = z

                r2 = x * x + y * y + z * z
                rn = _rsqrt16(jnp.maximum(r2, 1e-24))
                ux = x * rn
                uy = y * rn
                uz = z * rn
                shb = shbs[b]
                shb[1, sl] = SQ3 * ux
                shb[2, sl] = SQ3 * uy
                shb[3, sl] = SQ3 * uz

                x2 = ux * ux
                y2 = uy * uy
                z2 = uz * uz
                x2z2 = x2 + z2
                sh20 = SQ15 * ux * uz
                sh24 = (SQ15 / 2.0) * (z2 - x2)
                shb[4, sl] = sh20
                shb[5, sl] = SQ15 * ux * uy
                shb[6, sl] = SQ5 * (y2 - 0.5 * x2z2)
                shb[7, sl] = SQ15 * uy * uz
                shb[8, sl] = sh24
                fy = 4.0 * y2 - x2z2
                shb[9, sl] = C3A * (sh20 * uz + sh24 * ux)
                shb[10, sl] = SQ7 * sh20 * uy
                shb[11, sl] = C3B * fy * ux
                shb[12, sl] = C3C * uy * (2.0 * y2 - 3.0 * x2z2)
                shb[13, sl] = C3B * uz * fy
                shb[14, sl] = SQ7 * sh24 * uy
                shb[15, sl] = C3A * (sh24 * uz - sh20 * ux)
            pltpu.async_copy(vecbs[b], vec_out.at[:, pl.ds(e0, CH4)], osems[b])
            pltpu.async_copy(shbs[b], sh_out.at[:, pl.ds(e0, CH4)], osems[b])

            @pl.when(k + 2 < N4)
            def _(k=k, b=b):
                start_in4(k + 2, b)
        return 0

    lax.fori_loop(0, N4 // 2, pair4, 0)
    for b in (0, 1):
        wait_out4(N4 - 2 + b, b)


_sc_call = functools.partial(
    pl.kernel,
    mesh=plsc.VectorSubcoreMesh(core_axis_name="c", subcore_axis_name="s"),
    compiler_params=pltpu.CompilerParams(needs_layout_passes=False),
    out_type=[
        jax.ShapeDtypeStruct((3, N_EDGES), jnp.float32),
        jax.ShapeDtypeStruct((16, N_EDGES), jnp.float32),
        jax.ShapeDtypeStruct((2 * N_EDGES,), jnp.float32),
    ],
    scratch_types=[
        pltpu.VMEM((N_NODES,), jnp.float32),
        pltpu.VMEM((CH13,), jnp.int32),
        pltpu.VMEM((CH13,), jnp.int32),
        pltpu.VMEM((CH13,), jnp.int32),
        pltpu.VMEM((CH13,), jnp.int32),
        pltpu.VMEM((CH13,), jnp.float32),
        pltpu.VMEM((CH13,), jnp.float32),
        pltpu.VMEM((3, CH4), jnp.float32),
        pltpu.VMEM((3, CH4), jnp.float32),
        pltpu.VMEM((16, CH4), jnp.float32),
        pltpu.VMEM((16, CH4), jnp.float32),
        pltpu.SemaphoreType.DMA,
        pltpu.SemaphoreType.DMA,
        pltpu.SemaphoreType.DMA,
        pltpu.SemaphoreType.DMA,
    ],
)(_body)


def kernel(pos, edge_index):
    vec3, sh16, _ = _sc_call(pos[:, 0], pos[:, 1], pos[:, 2],
                             edge_index[0], edge_index[1])
    return vec3.T, sh16.T
